# baseline (speedup 1.0000x reference)
import functools

import jax
import jax.numpy as jnp
from jax import lax
from jax.experimental import pallas as pl
from jax.experimental.pallas import tpu as pltpu
from jax.experimental.pallas import tpu_sc as plsc

_B, _N, _C = 4, 1024, 768
_T = _B * _N
_NW = 32
_RPW = _T // _NW
_WPB = _N // _RPW
_NG = _RPW // 16
_MAXBLK = _NG + 1


def _sc_route_x1(im1f, im2f, m1, thr):
    mesh = plsc.VectorSubcoreMesh(core_axis_name="c", subcore_axis_name="s")

    @functools.partial(
        pl.kernel,
        out_type=jax.ShapeDtypeStruct((_T, _C), jnp.float32),
        mesh=mesh,
        scratch_types=[
            pltpu.VMEM((_MAXBLK * 16, _C), jnp.float32),
            pltpu.VMEM((_RPW + 16,), jnp.int32),
            pltpu.VMEM((_RPW + 16,), jnp.int32),
            pltpu.VMEM((_MAXBLK, 16), jnp.int32),
            pltpu.VMEM((_MAXBLK, 16), jnp.int32),
            pltpu.VMEM((_RPW,), jnp.float32),
            pltpu.VMEM((16,), jnp.float32),
            pltpu.SemaphoreType.DMA,
            pltpu.SemaphoreType.DMA,
            pltpu.SemaphoreType.DMA,
        ],
        compiler_params=pltpu.CompilerParams(use_tc_tiling_on_sc=True),
    )
    def k(im1_hbm, im2_hbm, m1_hbm, thr_hbm, x1_hbm,
          xbuf, ia1, ib1, ia2, ib2, mv, tv, semga, semgb, sems):
        wid = lax.axis_index("s") * 2 + lax.axis_index("c")
        t0 = wid * _RPW
        pltpu.sync_copy(
            m1_hbm.at[wid // _WPB, pl.ds((wid % _WPB) * _RPW, _RPW)], mv)
        pltpu.sync_copy(thr_hbm, tv)
        tvec = tv[...]

        lane = lax.iota(jnp.int32, 16)
        zero16 = jnp.full((16,), 0, jnp.int32)

        cnt_a = jnp.int32(0)
        cnt_b = jnp.int32(0)
        for g in range(_NG):
            keep = mv[pl.ds(g * 16, 16)] >= tvec
            ki = jnp.where(keep, jnp.full((16,), 1, jnp.int32), zero16)
            for j in range(16):
                kj = ki[j]
                tj = t0 + g * 16 + j

                @pl.when(kj > 0)
                def _():
                    ia1[pl.ds(cnt_a, 16)] = jnp.full((16,), tj, jnp.int32)

                @pl.when(kj == 0)
                def _():
                    ib1[pl.ds(cnt_b, 16)] = jnp.full((16,), tj, jnp.int32)

                cnt_a = cnt_a + kj
                cnt_b = cnt_b + (1 - kj)

        for blk in range(_MAXBLK):
            ia2[blk, :] = ia1[pl.ds(blk * 16, 16)]
            ib2[blk, :] = ib1[pl.ds(blk * 16, 16)]

        nblk_a = (cnt_a + 15) // 16
        nblk_b = (cnt_b + 15) // 16

        def gat_a(blk, carry):
            pltpu.async_copy(
                im1_hbm.at[ia2.at[blk]], xbuf.at[pl.ds(blk * 16, 16), :],
                semga)
            return carry

        lax.fori_loop(0, nblk_a, gat_a, 0)

        def gat_b(blk, carry):
            pltpu.async_copy(
                im2_hbm.at[ib2.at[blk]],
                xbuf.at[pl.ds((nblk_a + blk) * 16, 16), :], semgb)
            return carry

        lax.fori_loop(0, nblk_b, gat_b, 0)

        def drain_ga(i, carry):
            pltpu.make_async_copy(
                im1_hbm.at[pl.ds(0, 16), :], xbuf.at[pl.ds(0, 16), :],
                semga).wait()
            return carry

        lax.fori_loop(0, nblk_a, drain_ga, 0)

        def sca_a(blk, carry):
            pltpu.async_copy(
                xbuf.at[pl.ds(blk * 16, 16), :], x1_hbm.at[ia2.at[blk]],
                sems)
            return carry

        lax.fori_loop(0, nblk_a, sca_a, 0)

        def drain_gb(i, carry):
            pltpu.make_async_copy(
                im1_hbm.at[pl.ds(0, 16), :], xbuf.at[pl.ds(0, 16), :],
                semgb).wait()
            return carry

        lax.fori_loop(0, nblk_b, drain_gb, 0)

        def sca_b(blk, carry):
            pltpu.async_copy(
                xbuf.at[pl.ds((nblk_a + blk) * 16, 16), :],
                x1_hbm.at[ib2.at[blk]], sems)
            return carry

        lax.fori_loop(0, nblk_b, sca_b, 0)

        def drain_s(i, carry):
            pltpu.make_async_copy(
                im1_hbm.at[pl.ds(0, 16), :], xbuf.at[pl.ds(0, 16), :],
                sems).wait()
            return carry

        lax.fori_loop(0, nblk_a + nblk_b, drain_s, 0)

    return k(im1f, im2f, m1, thr).reshape(_B, _N, _C)


_BLKN = 512


def _tc_body(thr_ref, m2_ref, a_ref, b_ref, x2_ref):
    t = thr_ref[0]
    k2 = m2_ref[...] >= t
    x2_ref[...] = jnp.where(k2, b_ref[...], a_ref[...])


def _tc_select_x2(im1, im2, m2col, thr):
    grid = (_B, _N // _BLKN)
    return pl.pallas_call(
        _tc_body,
        grid=grid,
        in_specs=[
            pl.BlockSpec(memory_space=pltpu.SMEM),
            pl.BlockSpec((1, _BLKN, 1), lambda i, j: (i, j, 0)),
            pl.BlockSpec((1, _BLKN, _C), lambda i, j: (i, j, 0)),
            pl.BlockSpec((1, _BLKN, _C), lambda i, j: (i, j, 0)),
        ],
        out_specs=pl.BlockSpec((1, _BLKN, _C), lambda i, j: (i, j, 0)),
        out_shape=jax.ShapeDtypeStruct((_B, _N, _C), jnp.float32),
        compiler_params=pltpu.CompilerParams(
            dimension_semantics=("arbitrary", "arbitrary")),
    )(thr, m2col, im1, im2)


def kernel(im1, im2, mask1, mask2, mask_threshold):
    m2col = mask2.reshape(_B, _N, 1)
    thr16 = jnp.full((16,), mask_threshold, jnp.float32)
    thr1 = jnp.full((1,), mask_threshold, jnp.float32)
    x1 = _sc_route_x1(im1.reshape(_T, _C), im2.reshape(_T, _C), mask1, thr16)
    x2 = _tc_select_x2(im1, im2, m2col, thr1)
    return x1, x2
